# baseline (speedup 1.0000x reference)
import functools

import jax
import jax.numpy as jnp
from jax import lax
from jax.experimental import pallas as pl
from jax.experimental.pallas import tpu as pltpu
from jax.experimental.pallas import tpu_sc as plsc

N_NODES = 10000
N_EDGES = 160000
T = 4
D_IN = 8
D_EXOG = 16
HID = 64

NB = 512
NPAD = 10240


def _relu(v):
    return jnp.maximum(v, 0.0)


def _dot(a, b):
    return jnp.dot(a, b, preferred_element_type=jnp.float32)


def _stage1_body(x_ref, uw_ref,
                 ec_Wx, ec_bx, ec_Wu, ec_bu, ec_Wo, ec_bo, ec_Wc,
                 t1_W, t1_b, t2_W, t2_b, skip_W, skip_b,
                 gat_W, a_src, a_dst,
                 h_out, z_out, hg_out, ssrc_out, sdst_out):
    x2 = x_ref[...].reshape(T * NB, D_IN)
    u2 = uw_ref[...].reshape(T * NB, D_EXOG)
    cond = _dot(u2, ec_Wu[...]) + ec_bu[...]
    pre = _relu(_dot(x2, ec_Wx[...]) + ec_bx[...] + cond)
    h2 = _dot(pre, ec_Wo[...]) + ec_bo[...] + _dot(cond, ec_Wc[...])
    h4 = h2.reshape(T, NB, HID)
    h_out[...] = h4

    z1 = []
    for t in range(T):
        acc = t1_b[...]
        for k in range(3):
            j = t - 2 + k
            if 0 <= j < T:
                acc = acc + _dot(h4[j], t1_W[k])
        z1.append(_relu(acc))
    z2 = []
    for t in range(T):
        acc = t2_b[...]
        for k in range(3):
            j = t - 2 + k
            if 0 <= j < T:
                acc = acc + _dot(z1[j], t2_W[k])
        z2.append(_relu(acc))
    for t in range(T):
        z_t = _dot(h4[t], skip_W[...]) + skip_b[...] + z2[t]
        z_out[t] = z_t
        hg_t = _dot(z_t, gat_W[...])
        hg_out[t] = hg_t
        ssrc_out[t, :] = jnp.sum(hg_t * a_src[...], axis=-1)
        sdst_out[t, :] = jnp.sum(hg_t * a_dst[...], axis=-1)


def _stage1(x, uw, ec_Wx, ec_bx, ec_Wu, ec_bu, ec_Wo, ec_bo, ec_Wc,
            t1_W, t1_b, t2_W, t2_b, skip_W, skip_b, gat_W, a_src, a_dst,
            interpret=False):
    nblk = NPAD // NB
    row_spec = pl.BlockSpec((T, NB, HID), lambda i: (0, i, 0))
    vec_spec = pl.BlockSpec((T, NB), lambda i: (0, i))

    def full(shape):
        return pl.BlockSpec(shape, lambda i: tuple(0 for _ in shape))

    out_shapes = (
        jax.ShapeDtypeStruct((T, NPAD, HID), jnp.float32),
        jax.ShapeDtypeStruct((T, NPAD, HID), jnp.float32),
        jax.ShapeDtypeStruct((T, NPAD, HID), jnp.float32),
        jax.ShapeDtypeStruct((T, NPAD), jnp.float32),
        jax.ShapeDtypeStruct((T, NPAD), jnp.float32),
    )
    in_specs = [
        pl.BlockSpec((T, NB, D_IN), lambda i: (0, i, 0)),
        pl.BlockSpec((T, NB, D_EXOG), lambda i: (0, i, 0)),
        full((D_IN, HID)), full((1, HID)), full((D_EXOG, HID)), full((1, HID)),
        full((HID, HID)), full((1, HID)), full((HID, HID)),
        full((3, HID, HID)), full((1, HID)), full((3, HID, HID)), full((1, HID)),
        full((HID, HID)), full((1, HID)),
        full((HID, HID)), full((1, HID)), full((1, HID)),
    ]
    out_specs = (row_spec, row_spec, row_spec, vec_spec, vec_spec)
    return pl.pallas_call(
        _stage1_body,
        grid=(nblk,),
        in_specs=in_specs,
        out_specs=out_specs,
        out_shape=out_shapes,
        interpret=interpret,
    )(x, uw, ec_Wx, ec_bx.reshape(1, HID), ec_Wu, ec_bu.reshape(1, HID),
      ec_Wo, ec_bo.reshape(1, HID), ec_Wc,
      t1_W, t1_b.reshape(1, HID), t2_W, t2_b.reshape(1, HID),
      skip_W, skip_b.reshape(1, HID), gat_W,
      a_src.reshape(1, HID), a_dst.reshape(1, HID))


def _stage2_body(num_ref, den_ref, h_ref, z_ref, uh_ref,
                 gat_b, ro_Wx, ro_bx, ro_Wu, ro_bu, ro_Wo, ro_bo, ro_Wc,
                 out_ref):
    num = num_ref[0] + num_ref[1]
    den = den_ref[0]
    for w in range(1, den_ref.shape[0]):
        den = den + den_ref[w]
    s = _relu(num / (den[..., None] + 1e-16) + gat_b[...])
    hf = h_ref[...] + z_ref[...] + s
    wo = ro_Wo[0, 0]
    wc = ro_Wc[0, 0]
    bx = ro_bx[0, 0]
    bu = ro_bu[0, 0]
    bo = ro_bo[0, 0]
    for t in range(T):
        cond = jnp.sum(uh_ref[t] * ro_Wu[...], axis=-1) + bu
        pre = _relu(jnp.sum(hf[t] * ro_Wx[...], axis=-1) + bx + cond)
        out_ref[t, :] = pre * wo + bo + cond * wc


def _stage2(num, den, h, z, uh, gat_b, ro_Wx, ro_bx, ro_Wu, ro_bu,
            ro_Wo, ro_bo, ro_Wc, interpret=False):
    nblk = NPAD // NB
    nden = den.shape[0]

    def full(shape):
        return pl.BlockSpec(shape, lambda i: tuple(0 for _ in shape))

    in_specs = [
        pl.BlockSpec((2, T, NB, HID), lambda i: (0, 0, i, 0)),
        pl.BlockSpec((nden, T, NB), lambda i: (0, 0, i)),
        pl.BlockSpec((T, NB, HID), lambda i: (0, i, 0)),
        pl.BlockSpec((T, NB, HID), lambda i: (0, i, 0)),
        pl.BlockSpec((T, NB, D_EXOG), lambda i: (0, i, 0)),
        full((1, HID)),
        full((1, HID)),
        full((1, 1)), full((1, D_EXOG)), full((1, 1)),
        full((1, 1)), full((1, 1)), full((1, 1)),
    ]
    return pl.pallas_call(
        _stage2_body,
        grid=(nblk,),
        in_specs=in_specs,
        out_specs=pl.BlockSpec((T, NB), lambda i: (0, i)),
        out_shape=jax.ShapeDtypeStruct((T, NPAD), jnp.float32),
        interpret=interpret,
    )(num, den, h, z, uh, gat_b.reshape(1, HID), ro_Wx.reshape(1, HID),
      ro_bx.reshape(1, 1), ro_Wu.reshape(1, D_EXOG), ro_bu.reshape(1, 1),
      ro_Wo.reshape(1, 1), ro_bo.reshape(1, 1), ro_Wc.reshape(1, 1))


NW = 32
NQ = 4
GQ = 128
G = NQ * GQ
CHUNK = 5120
NBATCH = CHUNK // G
EP = NW * CHUNK
NP = N_NODES + 8
ROWS_PER_TILE = NPAD // 16


def _sc_edge_body(src_hbm, dst_hbm, stab_hbm, dtab_hbm, gmax_hbm, hgt_hbm,
                  num_hbm, den_hbm,
                  sidx, didx, stab, dtab, gvec, wbuf, gidx, rows, den,
                  num_sh, sem, sem2):
    sc = lax.axis_index("c")
    tid = lax.axis_index("s")
    wid = sc * 16 + tid
    zeros16 = jnp.zeros((16,), jnp.float32)

    pltpu.sync_copy(src_hbm.at[wid], sidx)
    pltpu.sync_copy(dst_hbm.at[wid], didx)

    for t in range(T):
        pltpu.sync_copy(stab_hbm.at[t], stab)
        pltpu.sync_copy(dtab_hbm.at[t], dtab)
        pltpu.sync_copy(gmax_hbm.at[t], gvec)
        g0 = gvec[...][0]

        @plsc.parallel_loop(0, NPAD // 16, unroll=8)
        def _zd(i):
            den[pl.ds(i * 16, 16)] = zeros16

        @plsc.parallel_loop(0, GQ, unroll=4)
        def _zr(r):
            for c in range(HID // 16):
                rows[0, r, pl.ds(c * 16, 16)] = zeros16
        for jz in range(ROWS_PER_TILE // GQ):
            pltpu.sync_copy(
                rows.at[0],
                num_sh.at[pl.ds(tid * ROWS_PER_TILE + jz * GQ, GQ)])
        plsc.subcore_barrier()

        izeros = jnp.zeros((16,), jnp.int32)

        def _batch(j, _):
            for q in range(NQ):
                @plsc.parallel_loop(0, GQ // 16, unroll=4)
                def _pa(g):
                    off = g * 16
                    sv = sidx[j, q, pl.ds(off, 16)]
                    dv = didx[j, q, pl.ds(off, 16)]
                    a = plsc.load_gather(stab, [sv])
                    b = plsc.load_gather(dtab, [dv])
                    e = a + b
                    e = jnp.where(e >= 0.0, e, 0.2 * e)
                    ub = b + g0
                    ub = jnp.where(ub >= 0.0, ub, 0.2 * ub)
                    w = jnp.exp(e - ub)
                    wbuf[q, pl.ds(off, 16)] = w
                    plsc.addupdate_scatter(den, [dv], w)
                    gidx[q, pl.ds(off, 16)] = sv + t * N_NODES

            copies = [pltpu.async_copy(hgt_hbm.at[gidx.at[q]], rows.at[q], sem)
                      for q in range(NQ)]
            scat = []
            for q in range(NQ):
                copies[q].wait()
                qv = izeros + q

                @plsc.parallel_loop(0, GQ, unroll=8)
                def _sc_rows(r):
                    wsplat = plsc.load_gather(wbuf, [qv, izeros + r])
                    for c in range(HID // 16):
                        rows[q, r, pl.ds(c * 16, 16)] = (
                            wsplat * rows[q, r, pl.ds(c * 16, 16)])

                scat.append(pltpu.async_copy(
                    rows.at[q], num_sh.at[didx.at[j, q]], sem2, add=True))
            for cp in scat:
                cp.wait()
            return _
        lax.fori_loop(0, NBATCH, _batch, None)

        plsc.subcore_barrier()
        pltpu.sync_copy(
            num_sh.at[pl.ds(tid * ROWS_PER_TILE, ROWS_PER_TILE)],
            num_hbm.at[sc, t, pl.ds(tid * ROWS_PER_TILE, ROWS_PER_TILE)])
        pltpu.sync_copy(den, den_hbm.at[wid, t])


def _gat_edges_sc(hg, ssrc, sdst, gmax, src, dst):
    npad_e = EP - N_EDGES
    srcp = jnp.concatenate(
        [src, jnp.full((npad_e,), N_NODES, jnp.int32)]).reshape(
            NW, NBATCH, NQ, GQ)
    dstp = jnp.concatenate(
        [dst, jnp.zeros((npad_e,), jnp.int32)]).reshape(NW, NBATCH, NQ, GQ)
    neg = jnp.full((T, NP - N_NODES), -1e30, jnp.float32)
    stab = jnp.concatenate([ssrc[:, :N_NODES], neg], axis=1)
    dtab = jnp.concatenate([sdst[:, :N_NODES], neg], axis=1)
    gmax16 = jnp.broadcast_to(gmax.reshape(T, 1), (T, 16))
    hgt = jnp.concatenate(
        [hg[:, :N_NODES].reshape(T * N_NODES, HID),
         jnp.zeros((8, HID), jnp.float32)])

    mesh = plsc.VectorSubcoreMesh(core_axis_name="c", subcore_axis_name="s")
    num, den = pl.kernel(
        _sc_edge_body,
        out_type=(
            jax.ShapeDtypeStruct((2, T, NPAD, HID), jnp.float32),
            jax.ShapeDtypeStruct((NW, T, NPAD), jnp.float32),
        ),
        mesh=mesh,
        compiler_params=pltpu.CompilerParams(
            needs_layout_passes=False, use_tc_tiling_on_sc=False),
        scratch_types=[
            pltpu.VMEM((NBATCH, NQ, GQ), jnp.int32),
            pltpu.VMEM((NBATCH, NQ, GQ), jnp.int32),
            pltpu.VMEM((NP,), jnp.float32),
            pltpu.VMEM((NP,), jnp.float32),
            pltpu.VMEM((16,), jnp.float32),
            pltpu.VMEM((NQ, GQ), jnp.float32),
            pltpu.VMEM((NQ, GQ), jnp.int32),
            pltpu.VMEM((NQ, GQ, HID), jnp.float32),
            pltpu.VMEM((NPAD,), jnp.float32),
            pltpu.VMEM_SHARED((NPAD, HID), jnp.float32),
            pltpu.SemaphoreType.DMA,
            pltpu.SemaphoreType.DMA,
        ],
    )(srcp, dstp, stab, dtab, gmax16, hgt)
    return num, den


def _gat_edges_xla(hg, ssrc, sdst, ub, src, dst):
    num = jnp.zeros((T, NPAD, HID), jnp.float32)
    den = jnp.zeros((T, NPAD), jnp.float32)
    for t in range(T):
        e = ssrc[t, src] + sdst[t, dst]
        e = jnp.where(e >= 0, e, 0.2 * e)
        w = jnp.exp(e - ub[t, dst])
        den = den.at[t].set(
            jax.ops.segment_sum(w, dst, num_segments=NPAD))
        msg = hg[t, src] * w[:, None]
        num = num.at[t].set(
            jax.ops.segment_sum(msg, dst, num_segments=NPAD))
    num2 = jnp.stack([num, jnp.zeros_like(num)])
    return num2, den[None]


def kernel(x, u_w, u_h, edge_index, ec_Wx, ec_bx, ec_Wu, ec_bu, ec_Wo,
           ec_bo, ec_Wc, t1_W, t1_b, t2_W, t2_b, skip_W, skip_b, gat_W,
           gat_b, a_src, a_dst, ro_Wx, ro_bx, ro_Wu, ro_bu, ro_Wo, ro_bo,
           ro_Wc, interpret=False):
    pad = NPAD - N_NODES
    x3 = jnp.pad(x[0], ((0, 0), (0, pad), (0, 0)))
    uw3 = jnp.pad(u_w[0], ((0, 0), (0, pad), (0, 0)))
    uh3 = jnp.pad(u_h[0], ((0, 0), (0, pad), (0, 0)))

    h, z, hg, ssrc, sdst = _stage1(
        x3, uw3, ec_Wx, ec_bx, ec_Wu, ec_bu, ec_Wo, ec_bo, ec_Wc,
        t1_W, t1_b, t2_W, t2_b, skip_W, skip_b, gat_W, a_src, a_dst,
        interpret=interpret)

    gmax = jnp.max(ssrc[:, :N_NODES], axis=1, keepdims=True)
    ubp = sdst + gmax
    ub = jnp.where(ubp >= 0, ubp, 0.2 * ubp)

    src = edge_index[0]
    dst = edge_index[1]
    if interpret:
        num2, den = _gat_edges_xla(hg, ssrc, sdst, ub, src, dst)
    else:
        num2, den = _gat_edges_sc(hg, ssrc, sdst, gmax, src, dst)

    out = _stage2(num2, den, h, z, uh3, gat_b, ro_Wx, ro_bx, ro_Wu,
                  ro_bu, ro_Wo, ro_bo, ro_Wc, interpret=interpret)
    return out[:, :N_NODES].reshape(1, T, N_NODES, 1)


# speedup vs baseline: 1.0209x; 1.0209x over previous
import functools

import jax
import jax.numpy as jnp
from jax import lax
from jax.experimental import pallas as pl
from jax.experimental.pallas import tpu as pltpu
from jax.experimental.pallas import tpu_sc as plsc

N_NODES = 10000
N_EDGES = 160000
T = 4
D_IN = 8
D_EXOG = 16
HID = 64

NB = 512
NPAD = 10240


def _relu(v):
    return jnp.maximum(v, 0.0)


def _dot(a, b):
    return jnp.dot(a, b, preferred_element_type=jnp.float32)


def _stage1_body(x_ref, uw_ref,
                 ec_Wx, ec_bx, ec_Wu, ec_bu, ec_Wo, ec_bo, ec_Wc,
                 t1_W, t1_b, t2_W, t2_b, skip_W, skip_b,
                 gat_W, a_src, a_dst,
                 h_out, z_out, hg_out, ssrc_out, sdst_out):
    x2 = x_ref[...].reshape(T * NB, D_IN)
    u2 = uw_ref[...].reshape(T * NB, D_EXOG)
    cond = _dot(u2, ec_Wu[...]) + ec_bu[...]
    pre = _relu(_dot(x2, ec_Wx[...]) + ec_bx[...] + cond)
    h2 = _dot(pre, ec_Wo[...]) + ec_bo[...] + _dot(cond, ec_Wc[...])
    h4 = h2.reshape(T, NB, HID)
    h_out[...] = h4

    z1 = []
    for t in range(T):
        acc = t1_b[...]
        for k in range(3):
            j = t - 2 + k
            if 0 <= j < T:
                acc = acc + _dot(h4[j], t1_W[k])
        z1.append(_relu(acc))
    z2 = []
    for t in range(T):
        acc = t2_b[...]
        for k in range(3):
            j = t - 2 + k
            if 0 <= j < T:
                acc = acc + _dot(z1[j], t2_W[k])
        z2.append(_relu(acc))
    for t in range(T):
        z_t = _dot(h4[t], skip_W[...]) + skip_b[...] + z2[t]
        z_out[t] = z_t
        hg_t = _dot(z_t, gat_W[...])
        hg_out[t] = hg_t
        ssrc_out[t, :] = jnp.sum(hg_t * a_src[...], axis=-1)
        sdst_out[t, :] = jnp.sum(hg_t * a_dst[...], axis=-1)


def _stage1(x, uw, ec_Wx, ec_bx, ec_Wu, ec_bu, ec_Wo, ec_bo, ec_Wc,
            t1_W, t1_b, t2_W, t2_b, skip_W, skip_b, gat_W, a_src, a_dst,
            interpret=False):
    nblk = NPAD // NB
    row_spec = pl.BlockSpec((T, NB, HID), lambda i: (0, i, 0))
    vec_spec = pl.BlockSpec((T, NB), lambda i: (0, i))

    def full(shape):
        return pl.BlockSpec(shape, lambda i: tuple(0 for _ in shape))

    out_shapes = (
        jax.ShapeDtypeStruct((T, NPAD, HID), jnp.float32),
        jax.ShapeDtypeStruct((T, NPAD, HID), jnp.float32),
        jax.ShapeDtypeStruct((T, NPAD, HID), jnp.float32),
        jax.ShapeDtypeStruct((T, NPAD), jnp.float32),
        jax.ShapeDtypeStruct((T, NPAD), jnp.float32),
    )
    in_specs = [
        pl.BlockSpec((T, NB, D_IN), lambda i: (0, i, 0)),
        pl.BlockSpec((T, NB, D_EXOG), lambda i: (0, i, 0)),
        full((D_IN, HID)), full((1, HID)), full((D_EXOG, HID)), full((1, HID)),
        full((HID, HID)), full((1, HID)), full((HID, HID)),
        full((3, HID, HID)), full((1, HID)), full((3, HID, HID)), full((1, HID)),
        full((HID, HID)), full((1, HID)),
        full((HID, HID)), full((1, HID)), full((1, HID)),
    ]
    out_specs = (row_spec, row_spec, row_spec, vec_spec, vec_spec)
    return pl.pallas_call(
        _stage1_body,
        grid=(nblk,),
        in_specs=in_specs,
        out_specs=out_specs,
        out_shape=out_shapes,
        interpret=interpret,
    )(x, uw, ec_Wx, ec_bx.reshape(1, HID), ec_Wu, ec_bu.reshape(1, HID),
      ec_Wo, ec_bo.reshape(1, HID), ec_Wc,
      t1_W, t1_b.reshape(1, HID), t2_W, t2_b.reshape(1, HID),
      skip_W, skip_b.reshape(1, HID), gat_W,
      a_src.reshape(1, HID), a_dst.reshape(1, HID))


def _stage2_body(num_ref, den_ref, h_ref, z_ref, uh_ref,
                 gat_b, ro_Wx, ro_bx, ro_Wu, ro_bu, ro_Wo, ro_bo, ro_Wc,
                 out_ref):
    num = num_ref[0] + num_ref[1]
    den = den_ref[0]
    for w in range(1, den_ref.shape[0]):
        den = den + den_ref[w]
    s = _relu(num / (den[..., None] + 1e-16) + gat_b[...])
    hf = h_ref[...] + z_ref[...] + s
    wo = ro_Wo[0, 0]
    wc = ro_Wc[0, 0]
    bx = ro_bx[0, 0]
    bu = ro_bu[0, 0]
    bo = ro_bo[0, 0]
    for t in range(T):
        cond = jnp.sum(uh_ref[t] * ro_Wu[...], axis=-1) + bu
        pre = _relu(jnp.sum(hf[t] * ro_Wx[...], axis=-1) + bx + cond)
        out_ref[t, :] = pre * wo + bo + cond * wc


def _stage2(num, den, h, z, uh, gat_b, ro_Wx, ro_bx, ro_Wu, ro_bu,
            ro_Wo, ro_bo, ro_Wc, interpret=False):
    nblk = NPAD // NB
    nden = den.shape[0]

    def full(shape):
        return pl.BlockSpec(shape, lambda i: tuple(0 for _ in shape))

    in_specs = [
        pl.BlockSpec((2, T, NB, HID), lambda i: (0, 0, i, 0)),
        pl.BlockSpec((nden, T, NB), lambda i: (0, 0, i)),
        pl.BlockSpec((T, NB, HID), lambda i: (0, i, 0)),
        pl.BlockSpec((T, NB, HID), lambda i: (0, i, 0)),
        pl.BlockSpec((T, NB, D_EXOG), lambda i: (0, i, 0)),
        full((1, HID)),
        full((1, HID)),
        full((1, 1)), full((1, D_EXOG)), full((1, 1)),
        full((1, 1)), full((1, 1)), full((1, 1)),
    ]
    return pl.pallas_call(
        _stage2_body,
        grid=(nblk,),
        in_specs=in_specs,
        out_specs=pl.BlockSpec((T, NB), lambda i: (0, i)),
        out_shape=jax.ShapeDtypeStruct((T, NPAD), jnp.float32),
        interpret=interpret,
    )(num, den, h, z, uh, gat_b.reshape(1, HID), ro_Wx.reshape(1, HID),
      ro_bx.reshape(1, 1), ro_Wu.reshape(1, D_EXOG), ro_bu.reshape(1, 1),
      ro_Wo.reshape(1, 1), ro_bo.reshape(1, 1), ro_Wc.reshape(1, 1))


NW = 32
NQ = 2
GQ = 128
G = NQ * GQ
CHUNK = 5120
NBATCH = CHUNK // G
EP = NW * CHUNK
NP = N_NODES + 8
ROWS_PER_TILE = NPAD // 16


def _sc_edge_body(src_hbm, dst_hbm, stab_hbm, dtab_hbm, gmax_hbm, hgt_hbm,
                  num_hbm, den_hbm,
                  sidx, didx, stab, dtab, gvec, wbuf, gidx, rows, den,
                  num_sh, sg00, sg01, sg10, sg11, sem2):
    sc = lax.axis_index("c")
    tid = lax.axis_index("s")
    wid = sc * 16 + tid
    zeros16 = jnp.zeros((16,), jnp.float32)
    izeros = jnp.zeros((16,), jnp.int32)
    sg = ((sg00, sg01), (sg10, sg11))

    pltpu.sync_copy(src_hbm.at[wid], sidx)
    pltpu.sync_copy(dst_hbm.at[wid], didx)

    for t in range(T):
        pltpu.sync_copy(stab_hbm.at[t], stab)
        pltpu.sync_copy(dtab_hbm.at[t], dtab)
        pltpu.sync_copy(gmax_hbm.at[t], gvec)
        g0 = gvec[...][0]

        @plsc.parallel_loop(0, NPAD // 16, unroll=8)
        def _zd(i):
            den[pl.ds(i * 16, 16)] = zeros16

        @plsc.parallel_loop(0, GQ, unroll=4)
        def _zr(r):
            for c in range(HID // 16):
                rows[0, 0, r, pl.ds(c * 16, 16)] = zeros16
        for jz in range(ROWS_PER_TILE // GQ):
            pltpu.sync_copy(
                rows.at[0, 0],
                num_sh.at[pl.ds(tid * ROWS_PER_TILE + jz * GQ, GQ)])
        plsc.subcore_barrier()

        def _pa(j, bank):
            for q in range(NQ):
                @plsc.parallel_loop(0, GQ // 16, unroll=4)
                def _paq(g):
                    off = g * 16
                    sv = sidx[j, q, pl.ds(off, 16)]
                    dv = didx[j, q, pl.ds(off, 16)]
                    a = plsc.load_gather(stab, [sv])
                    b = plsc.load_gather(dtab, [dv])
                    e = a + b
                    e = jnp.where(e >= 0.0, e, 0.2 * e)
                    ub = b + g0
                    ub = jnp.where(ub >= 0.0, ub, 0.2 * ub)
                    w = jnp.exp(e - ub)
                    wbuf[bank, q, pl.ds(off, 16)] = w
                    plsc.addupdate_scatter(den, [dv], w)
                    gidx[bank, q, pl.ds(off, 16)] = sv + t * N_NODES

        def _gat(bank):
            for q in range(NQ):
                pltpu.async_copy(hgt_hbm.at[gidx.at[bank, q]],
                                 rows.at[bank, q], sg[bank][q])

        def _wait_gat(bank):
            for q in range(NQ):
                pltpu.make_async_copy(hgt_hbm.at[gidx.at[bank, q]],
                                      rows.at[bank, q], sg[bank][q]).wait()

        def _scale(bank):
            for q in range(NQ):
                bv = izeros + bank
                qv = izeros + q

                @plsc.parallel_loop(0, GQ, unroll=8)
                def _sr(r):
                    wsplat = plsc.load_gather(wbuf, [bv, qv, izeros + r])
                    for c in range(HID // 16):
                        rows[bank, q, r, pl.ds(c * 16, 16)] = (
                            wsplat * rows[bank, q, r, pl.ds(c * 16, 16)])

        def _scat(j, bank):
            for q in range(NQ):
                pltpu.async_copy(rows.at[bank, q], num_sh.at[didx.at[j, q]],
                                 sem2, add=True)

        def _drain_scat(j, bank):
            for q in range(NQ):
                pltpu.make_async_copy(rows.at[bank, q],
                                      num_sh.at[didx.at[j, q]], sem2).wait()

        _pa(0, 0)
        _gat(0)

        def _jj(jj, _):
            j0 = 2 * jj
            j1 = j0 + 1
            _pa(j1, 1)

            @pl.when(jj > 0)
            def _():
                _drain_scat(j0 - 1, 1)
            _gat(1)
            _wait_gat(0)
            _scale(0)
            _scat(j0, 0)

            @pl.when(jj < NBATCH // 2 - 1)
            def _():
                _pa(j0 + 2, 0)
            _wait_gat(1)
            _scale(1)
            _drain_scat(j0, 0)

            @pl.when(jj < NBATCH // 2 - 1)
            def _():
                _gat(0)
            _scat(j1, 1)
            return _
        lax.fori_loop(0, NBATCH // 2, _jj, None)
        _drain_scat(NBATCH - 1, 1)

        plsc.subcore_barrier()
        pltpu.sync_copy(
            num_sh.at[pl.ds(tid * ROWS_PER_TILE, ROWS_PER_TILE)],
            num_hbm.at[sc, t, pl.ds(tid * ROWS_PER_TILE, ROWS_PER_TILE)])
        pltpu.sync_copy(den, den_hbm.at[wid, t])


def _gat_edges_sc(hg, ssrc, sdst, gmax, src, dst):
    npad_e = EP - N_EDGES
    srcp = jnp.concatenate(
        [src, jnp.full((npad_e,), N_NODES, jnp.int32)]).reshape(
            NW, NBATCH, NQ, GQ)
    dstp = jnp.concatenate(
        [dst, jnp.zeros((npad_e,), jnp.int32)]).reshape(NW, NBATCH, NQ, GQ)
    neg = jnp.full((T, NP - N_NODES), -1e30, jnp.float32)
    stab = jnp.concatenate([ssrc[:, :N_NODES], neg], axis=1)
    dtab = jnp.concatenate([sdst[:, :N_NODES], neg], axis=1)
    gmax16 = jnp.broadcast_to(gmax.reshape(T, 1), (T, 16))
    hgt = jnp.concatenate(
        [hg[:, :N_NODES].reshape(T * N_NODES, HID),
         jnp.zeros((8, HID), jnp.float32)])

    mesh = plsc.VectorSubcoreMesh(core_axis_name="c", subcore_axis_name="s")
    num, den = pl.kernel(
        _sc_edge_body,
        out_type=(
            jax.ShapeDtypeStruct((2, T, NPAD, HID), jnp.float32),
            jax.ShapeDtypeStruct((NW, T, NPAD), jnp.float32),
        ),
        mesh=mesh,
        compiler_params=pltpu.CompilerParams(
            needs_layout_passes=False, use_tc_tiling_on_sc=False),
        scratch_types=[
            pltpu.VMEM((NBATCH, NQ, GQ), jnp.int32),
            pltpu.VMEM((NBATCH, NQ, GQ), jnp.int32),
            pltpu.VMEM((NP,), jnp.float32),
            pltpu.VMEM((NP,), jnp.float32),
            pltpu.VMEM((16,), jnp.float32),
            pltpu.VMEM((2, NQ, GQ), jnp.float32),
            pltpu.VMEM((2, NQ, GQ), jnp.int32),
            pltpu.VMEM((2, NQ, GQ, HID), jnp.float32),
            pltpu.VMEM((NPAD,), jnp.float32),
            pltpu.VMEM_SHARED((NPAD, HID), jnp.float32),
            pltpu.SemaphoreType.DMA,
            pltpu.SemaphoreType.DMA,
            pltpu.SemaphoreType.DMA,
            pltpu.SemaphoreType.DMA,
            pltpu.SemaphoreType.DMA,
        ],
    )(srcp, dstp, stab, dtab, gmax16, hgt)
    return num, den


def _gat_edges_xla(hg, ssrc, sdst, ub, src, dst):
    num = jnp.zeros((T, NPAD, HID), jnp.float32)
    den = jnp.zeros((T, NPAD), jnp.float32)
    for t in range(T):
        e = ssrc[t, src] + sdst[t, dst]
        e = jnp.where(e >= 0, e, 0.2 * e)
        w = jnp.exp(e - ub[t, dst])
        den = den.at[t].set(
            jax.ops.segment_sum(w, dst, num_segments=NPAD))
        msg = hg[t, src] * w[:, None]
        num = num.at[t].set(
            jax.ops.segment_sum(msg, dst, num_segments=NPAD))
    num2 = jnp.stack([num, jnp.zeros_like(num)])
    return num2, den[None]


def kernel(x, u_w, u_h, edge_index, ec_Wx, ec_bx, ec_Wu, ec_bu, ec_Wo,
           ec_bo, ec_Wc, t1_W, t1_b, t2_W, t2_b, skip_W, skip_b, gat_W,
           gat_b, a_src, a_dst, ro_Wx, ro_bx, ro_Wu, ro_bu, ro_Wo, ro_bo,
           ro_Wc, interpret=False):
    pad = NPAD - N_NODES
    x3 = jnp.pad(x[0], ((0, 0), (0, pad), (0, 0)))
    uw3 = jnp.pad(u_w[0], ((0, 0), (0, pad), (0, 0)))
    uh3 = jnp.pad(u_h[0], ((0, 0), (0, pad), (0, 0)))

    h, z, hg, ssrc, sdst = _stage1(
        x3, uw3, ec_Wx, ec_bx, ec_Wu, ec_bu, ec_Wo, ec_bo, ec_Wc,
        t1_W, t1_b, t2_W, t2_b, skip_W, skip_b, gat_W, a_src, a_dst,
        interpret=interpret)

    gmax = jnp.max(ssrc[:, :N_NODES], axis=1, keepdims=True)
    ubp = sdst + gmax
    ub = jnp.where(ubp >= 0, ubp, 0.2 * ubp)

    src = edge_index[0]
    dst = edge_index[1]
    if interpret:
        num2, den = _gat_edges_xla(hg, ssrc, sdst, ub, src, dst)
    else:
        num2, den = _gat_edges_sc(hg, ssrc, sdst, gmax, src, dst)

    out = _stage2(num2, den, h, z, uh3, gat_b, ro_Wx, ro_bx, ro_Wu,
                  ro_bu, ro_Wo, ro_bo, ro_Wc, interpret=interpret)
    return out[:, :N_NODES].reshape(1, T, N_NODES, 1)


# speedup vs baseline: 1.1486x; 1.1252x over previous
import functools

import jax
import jax.numpy as jnp
from jax import lax
from jax.experimental import pallas as pl
from jax.experimental.pallas import tpu as pltpu
from jax.experimental.pallas import tpu_sc as plsc

N_NODES = 10000
N_EDGES = 160000
T = 4
D_IN = 8
D_EXOG = 16
HID = 64

NB = 400
NPAD = 10240


def _relu(v):
    return jnp.maximum(v, 0.0)


def _dot(a, b):
    return jnp.dot(a, b, preferred_element_type=jnp.float32)


def _stage1_body(x_ref, uw_ref,
                 ec_Wx, ec_bx, ec_Wu, ec_bu, ec_Wo, ec_bo, ec_Wc,
                 t1_W, t1_b, t2_W, t2_b, skip_W, skip_b,
                 gat_W, a_src, a_dst,
                 h_out, z_out, hg_out, ssrc_out, sdst_out):
    x2 = x_ref[...].reshape(T * NB, D_IN)
    u2 = uw_ref[...].reshape(T * NB, D_EXOG)
    cond = _dot(u2, ec_Wu[...]) + ec_bu[...]
    pre = _relu(_dot(x2, ec_Wx[...]) + ec_bx[...] + cond)
    h2 = _dot(pre, ec_Wo[...]) + ec_bo[...] + _dot(cond, ec_Wc[...])
    h4 = h2.reshape(T, NB, HID)
    h_out[...] = h4

    z1 = []
    for t in range(T):
        acc = t1_b[...]
        for k in range(3):
            j = t - 2 + k
            if 0 <= j < T:
                acc = acc + _dot(h4[j], t1_W[k])
        z1.append(_relu(acc))
    z2 = []
    for t in range(T):
        acc = t2_b[...]
        for k in range(3):
            j = t - 2 + k
            if 0 <= j < T:
                acc = acc + _dot(z1[j], t2_W[k])
        z2.append(_relu(acc))
    for t in range(T):
        z_t = _dot(h4[t], skip_W[...]) + skip_b[...] + z2[t]
        z_out[t] = z_t
        hg_t = _dot(z_t, gat_W[...])
        hg_out[t] = hg_t
        ssrc_out[:, t] = jnp.sum(hg_t * a_src[...], axis=-1)
        sdst_out[:, t] = jnp.sum(hg_t * a_dst[...], axis=-1)


def _stage1(x, uw, ec_Wx, ec_bx, ec_Wu, ec_bu, ec_Wo, ec_bo, ec_Wc,
            t1_W, t1_b, t2_W, t2_b, skip_W, skip_b, gat_W, a_src, a_dst,
            interpret=False):
    nblk = N_NODES // NB
    row_spec = pl.BlockSpec((T, NB, HID), lambda i: (0, i, 0))
    vec_spec = pl.BlockSpec((NB, T), lambda i: (i, 0))

    def full(shape):
        return pl.BlockSpec(shape, lambda i: tuple(0 for _ in shape))

    out_shapes = (
        jax.ShapeDtypeStruct((T, N_NODES, HID), jnp.float32),
        jax.ShapeDtypeStruct((T, N_NODES, HID), jnp.float32),
        jax.ShapeDtypeStruct((T, N_NODES, HID), jnp.float32),
        jax.ShapeDtypeStruct((N_NODES, T), jnp.float32),
        jax.ShapeDtypeStruct((N_NODES, T), jnp.float32),
    )
    in_specs = [
        pl.BlockSpec((T, NB, D_IN), lambda i: (0, i, 0)),
        pl.BlockSpec((T, NB, D_EXOG), lambda i: (0, i, 0)),
        full((D_IN, HID)), full((1, HID)), full((D_EXOG, HID)), full((1, HID)),
        full((HID, HID)), full((1, HID)), full((HID, HID)),
        full((3, HID, HID)), full((1, HID)), full((3, HID, HID)), full((1, HID)),
        full((HID, HID)), full((1, HID)),
        full((HID, HID)), full((1, HID)), full((1, HID)),
    ]
    out_specs = (row_spec, row_spec, row_spec, vec_spec, vec_spec)
    return pl.pallas_call(
        _stage1_body,
        grid=(nblk,),
        in_specs=in_specs,
        out_specs=out_specs,
        out_shape=out_shapes,
        interpret=interpret,
    )(x, uw, ec_Wx, ec_bx.reshape(1, HID), ec_Wu, ec_bu.reshape(1, HID),
      ec_Wo, ec_bo.reshape(1, HID), ec_Wc,
      t1_W, t1_b.reshape(1, HID), t2_W, t2_b.reshape(1, HID),
      skip_W, skip_b.reshape(1, HID), gat_W,
      a_src.reshape(1, HID), a_dst.reshape(1, HID))


def _stage2_body(num_ref, den_ref, h_ref, z_ref, uh_ref,
                 gat_b, ro_Wx, ro_bx, ro_Wu, ro_bu, ro_Wo, ro_bo, ro_Wc,
                 out_ref):
    num = num_ref[0] + num_ref[1]
    den = den_ref[...]
    wo = ro_Wo[0, 0]
    wc = ro_Wc[0, 0]
    bx = ro_bx[0, 0]
    bu = ro_bu[0, 0]
    bo = ro_bo[0, 0]
    for t in range(T):
        s = _relu(num[t] / (den[:, t][:, None] + 1e-16) + gat_b[...])
        hf = h_ref[t] + z_ref[t] + s
        cond = jnp.sum(uh_ref[t] * ro_Wu[...], axis=-1) + bu
        pre = _relu(jnp.sum(hf * ro_Wx[...], axis=-1) + bx + cond)
        out_ref[:, t] = pre * wo + bo + cond * wc


def _stage2(num, den, h, z, uh, gat_b, ro_Wx, ro_bx, ro_Wu, ro_bu,
            ro_Wo, ro_bo, ro_Wc, interpret=False):
    nblk = N_NODES // NB

    def full(shape):
        return pl.BlockSpec(shape, lambda i: tuple(0 for _ in shape))

    in_specs = [
        pl.BlockSpec((2, T, NB, HID), lambda i: (0, 0, i, 0)),
        pl.BlockSpec((NB, T), lambda i: (i, 0)),
        pl.BlockSpec((T, NB, HID), lambda i: (0, i, 0)),
        pl.BlockSpec((T, NB, HID), lambda i: (0, i, 0)),
        pl.BlockSpec((T, NB, D_EXOG), lambda i: (0, i, 0)),
        full((1, HID)),
        full((1, HID)),
        full((1, 1)), full((1, D_EXOG)), full((1, 1)),
        full((1, 1)), full((1, 1)), full((1, 1)),
    ]
    return pl.pallas_call(
        _stage2_body,
        grid=(nblk,),
        in_specs=in_specs,
        out_specs=pl.BlockSpec((NB, T), lambda i: (i, 0)),
        out_shape=jax.ShapeDtypeStruct((N_NODES, T), jnp.float32),
        interpret=interpret,
    )(num, den, h, z, uh, gat_b.reshape(1, HID), ro_Wx.reshape(1, HID),
      ro_bx.reshape(1, 1), ro_Wu.reshape(1, D_EXOG), ro_bu.reshape(1, 1),
      ro_Wo.reshape(1, 1), ro_bo.reshape(1, 1), ro_Wc.reshape(1, 1))


NW = 32
NQ = 2
GQ = 128
G = NQ * GQ
CHUNK = 5120
NBATCH = CHUNK // G
EP = NW * CHUNK
NP = N_NODES + 8
ROWS_PER_TILE = NPAD // 16


def _sc_edge_body(src_hbm, dst_hbm, stab_hbm, dtab_hbm, gmax_hbm, hgt_hbm,
                  num_hbm, den_hbm,
                  sidx, didx, stab, dtab, gvec, wbuf, gidx, rows, den,
                  num_sh, sg00, sg01, sg10, sg11, sem2):
    sc = lax.axis_index("c")
    tid = lax.axis_index("s")
    wid = sc * 16 + tid
    zeros16 = jnp.zeros((16,), jnp.float32)
    izeros = jnp.zeros((16,), jnp.int32)
    sg = ((sg00, sg01), (sg10, sg11))

    pltpu.sync_copy(src_hbm.at[wid], sidx)
    pltpu.sync_copy(dst_hbm.at[wid], didx)

    for t in range(T):
        pltpu.sync_copy(stab_hbm.at[t], stab)
        pltpu.sync_copy(dtab_hbm.at[t], dtab)
        pltpu.sync_copy(gmax_hbm.at[t], gvec)
        g0 = gvec[...][0]

        @plsc.parallel_loop(0, NPAD // 16, unroll=8)
        def _zd(i):
            den[pl.ds(i * 16, 16)] = zeros16

        @plsc.parallel_loop(0, GQ, unroll=4)
        def _zr(r):
            for c in range(HID // 16):
                rows[0, 0, r, pl.ds(c * 16, 16)] = zeros16
        for jz in range(ROWS_PER_TILE // GQ):
            pltpu.sync_copy(
                rows.at[0, 0],
                num_sh.at[pl.ds(tid * ROWS_PER_TILE + jz * GQ, GQ)])
        plsc.subcore_barrier()

        def _pa(j, bank):
            for q in range(NQ):
                @plsc.parallel_loop(0, GQ // 16, unroll=4)
                def _paq(g):
                    off = g * 16
                    sv = sidx[j, q, pl.ds(off, 16)]
                    dv = didx[j, q, pl.ds(off, 16)]
                    a = plsc.load_gather(stab, [sv])
                    b = plsc.load_gather(dtab, [dv])
                    e = a + b
                    e = jnp.where(e >= 0.0, e, 0.2 * e)
                    ub = b + g0
                    ub = jnp.where(ub >= 0.0, ub, 0.2 * ub)
                    w = jnp.exp(e - ub)
                    wbuf[bank, q, pl.ds(off, 16)] = w
                    plsc.addupdate_scatter(den, [dv], w)
                    gidx[bank, q, pl.ds(off, 16)] = (
                        jnp.minimum(sv, N_NODES - 1) + t * N_NODES)

        def _gat(bank):
            for q in range(NQ):
                pltpu.async_copy(hgt_hbm.at[gidx.at[bank, q]],
                                 rows.at[bank, q], sg[bank][q])

        def _wait_gat(bank):
            for q in range(NQ):
                pltpu.make_async_copy(hgt_hbm.at[gidx.at[bank, q]],
                                      rows.at[bank, q], sg[bank][q]).wait()

        def _scale(bank):
            for q in range(NQ):
                bv = izeros + bank
                qv = izeros + q

                @plsc.parallel_loop(0, GQ, unroll=8)
                def _sr(r):
                    wsplat = plsc.load_gather(wbuf, [bv, qv, izeros + r])
                    for c in range(HID // 16):
                        rows[bank, q, r, pl.ds(c * 16, 16)] = (
                            wsplat * rows[bank, q, r, pl.ds(c * 16, 16)])

        def _scat(j, bank):
            for q in range(NQ):
                pltpu.async_copy(rows.at[bank, q], num_sh.at[didx.at[j, q]],
                                 sem2, add=True)

        def _drain_scat(j, bank):
            for q in range(NQ):
                pltpu.make_async_copy(rows.at[bank, q],
                                      num_sh.at[didx.at[j, q]], sem2).wait()

        _pa(0, 0)
        _gat(0)

        def _jj(jj, _):
            j0 = 2 * jj
            j1 = j0 + 1
            _pa(j1, 1)

            @pl.when(jj > 0)
            def _():
                _drain_scat(j0 - 1, 1)
            _gat(1)
            _wait_gat(0)
            _scale(0)
            _scat(j0, 0)

            @pl.when(jj < NBATCH // 2 - 1)
            def _():
                _pa(j0 + 2, 0)
            _wait_gat(1)
            _scale(1)
            _drain_scat(j0, 0)

            @pl.when(jj < NBATCH // 2 - 1)
            def _():
                _gat(0)
            _scat(j1, 1)
            return _
        lax.fori_loop(0, NBATCH // 2, _jj, None)
        _drain_scat(NBATCH - 1, 1)

        plsc.subcore_barrier()
        pltpu.sync_copy(
            num_sh.at[pl.ds(tid * ROWS_PER_TILE, ROWS_PER_TILE)],
            num_hbm.at[sc, t, pl.ds(tid * ROWS_PER_TILE, ROWS_PER_TILE)])
        pltpu.sync_copy(den, den_hbm.at[wid, t])


def _gat_edges_sc(hg, ssrc, sdst, gmax, src, dst):
    npad_e = EP - N_EDGES
    srcp = jnp.concatenate(
        [src, jnp.full((npad_e,), N_NODES, jnp.int32)]).reshape(
            NW, NBATCH, NQ, GQ)
    dstp = jnp.concatenate(
        [dst, jnp.zeros((npad_e,), jnp.int32)]).reshape(NW, NBATCH, NQ, GQ)
    neg = jnp.full((T, NP - N_NODES), -1e30, jnp.float32)
    stab = jnp.concatenate([ssrc.T, neg], axis=1)
    dtab = jnp.concatenate([sdst.T, neg], axis=1)
    gmax16 = jnp.broadcast_to(gmax.reshape(T, 1), (T, 16))
    hgt = hg.reshape(T * N_NODES, HID)

    mesh = plsc.VectorSubcoreMesh(core_axis_name="c", subcore_axis_name="s")
    num, den = pl.kernel(
        _sc_edge_body,
        out_type=(
            jax.ShapeDtypeStruct((2, T, NPAD, HID), jnp.float32),
            jax.ShapeDtypeStruct((NW, T, NPAD), jnp.float32),
        ),
        mesh=mesh,
        compiler_params=pltpu.CompilerParams(
            needs_layout_passes=False, use_tc_tiling_on_sc=False),
        scratch_types=[
            pltpu.VMEM((NBATCH, NQ, GQ), jnp.int32),
            pltpu.VMEM((NBATCH, NQ, GQ), jnp.int32),
            pltpu.VMEM((NP,), jnp.float32),
            pltpu.VMEM((NP,), jnp.float32),
            pltpu.VMEM((16,), jnp.float32),
            pltpu.VMEM((2, NQ, GQ), jnp.float32),
            pltpu.VMEM((2, NQ, GQ), jnp.int32),
            pltpu.VMEM((2, NQ, GQ, HID), jnp.float32),
            pltpu.VMEM((NPAD,), jnp.float32),
            pltpu.VMEM_SHARED((NPAD, HID), jnp.float32),
            pltpu.SemaphoreType.DMA,
            pltpu.SemaphoreType.DMA,
            pltpu.SemaphoreType.DMA,
            pltpu.SemaphoreType.DMA,
            pltpu.SemaphoreType.DMA,
        ],
    )(srcp, dstp, stab, dtab, gmax16, hgt)
    return num, den


def _gat_edges_xla(hg, ssrc, sdst, gmax, src, dst):
    num = jnp.zeros((T, NPAD, HID), jnp.float32)
    den = jnp.zeros((T, NPAD), jnp.float32)
    for t in range(T):
        e = ssrc[src, t] + sdst[dst, t]
        e = jnp.where(e >= 0, e, 0.2 * e)
        ubv = sdst[dst, t] + gmax[t]
        ubv = jnp.where(ubv >= 0, ubv, 0.2 * ubv)
        w = jnp.exp(e - ubv)
        den = den.at[t].set(
            jax.ops.segment_sum(w, dst, num_segments=NPAD))
        msg = hg[t, src] * w[:, None]
        num = num.at[t].set(
            jax.ops.segment_sum(msg, dst, num_segments=NPAD))
    num2 = jnp.stack([num, jnp.zeros_like(num)])
    return num2, den[None]


def kernel(x, u_w, u_h, edge_index, ec_Wx, ec_bx, ec_Wu, ec_bu, ec_Wo,
           ec_bo, ec_Wc, t1_W, t1_b, t2_W, t2_b, skip_W, skip_b, gat_W,
           gat_b, a_src, a_dst, ro_Wx, ro_bx, ro_Wu, ro_bu, ro_Wo, ro_bo,
           ro_Wc, interpret=False):
    h, z, hg, ssrc, sdst = _stage1(
        x[0], u_w[0], ec_Wx, ec_bx, ec_Wu, ec_bu, ec_Wo, ec_bo, ec_Wc,
        t1_W, t1_b, t2_W, t2_b, skip_W, skip_b, gat_W, a_src, a_dst,
        interpret=interpret)

    gmax = jnp.max(ssrc, axis=0)

    src = edge_index[0]
    dst = edge_index[1]
    if interpret:
        num2, den = _gat_edges_xla(hg, ssrc, sdst, gmax, src, dst)
    else:
        num2, den = _gat_edges_sc(hg, ssrc, sdst, gmax, src, dst)

    denT = jnp.sum(den, axis=0)[:, :N_NODES].T

    out = _stage2(num2, denT, h, z, u_h[0], gat_b, ro_Wx, ro_bx, ro_Wu,
                  ro_bu, ro_Wo, ro_bo, ro_Wc, interpret=interpret)
    return out.T.reshape(1, T, N_NODES, 1)


# speedup vs baseline: 1.1735x; 1.0216x over previous
import functools

import jax
import jax.numpy as jnp
from jax import lax
from jax.experimental import pallas as pl
from jax.experimental.pallas import tpu as pltpu
from jax.experimental.pallas import tpu_sc as plsc

N_NODES = 10000
N_EDGES = 160000
T = 4
D_IN = 8
D_EXOG = 16
HID = 64

NB = 2000
NPAD = 10240


def _relu(v):
    return jnp.maximum(v, 0.0)


def _dot(a, b):
    return jnp.dot(a, b, preferred_element_type=jnp.float32)


def _stage1_body(x_ref, uw_ref,
                 ec_Wx, ec_bx, ec_Wu, ec_bu, ec_Wo, ec_bo, ec_Wc,
                 t1_W, t1_b, t2_W, t2_b, skip_W, skip_b,
                 gat_W, a_src, a_dst,
                 h_out, z_out, hg_out, ssrc_out, sdst_out):
    x2 = x_ref[...].reshape(T * NB, D_IN)
    u2 = uw_ref[...].reshape(T * NB, D_EXOG)
    cond = _dot(u2, ec_Wu[...]) + ec_bu[...]
    pre = _relu(_dot(x2, ec_Wx[...]) + ec_bx[...] + cond)
    h2 = _dot(pre, ec_Wo[...]) + ec_bo[...] + _dot(cond, ec_Wc[...])
    h4 = h2.reshape(T, NB, HID)
    h_out[...] = h4

    z1 = []
    for t in range(T):
        acc = t1_b[...]
        for k in range(3):
            j = t - 2 + k
            if 0 <= j < T:
                acc = acc + _dot(h4[j], t1_W[k])
        z1.append(_relu(acc))
    z2 = []
    for t in range(T):
        acc = t2_b[...]
        for k in range(3):
            j = t - 2 + k
            if 0 <= j < T:
                acc = acc + _dot(z1[j], t2_W[k])
        z2.append(_relu(acc))
    for t in range(T):
        z_t = _dot(h4[t], skip_W[...]) + skip_b[...] + z2[t]
        z_out[t] = z_t
        hg_t = _dot(z_t, gat_W[...])
        hg_out[t] = hg_t
        ssrc_out[:, t] = jnp.sum(hg_t * a_src[...], axis=-1)
        sdst_out[:, t] = jnp.sum(hg_t * a_dst[...], axis=-1)


def _stage1(x, uw, ec_Wx, ec_bx, ec_Wu, ec_bu, ec_Wo, ec_bo, ec_Wc,
            t1_W, t1_b, t2_W, t2_b, skip_W, skip_b, gat_W, a_src, a_dst,
            interpret=False):
    nblk = N_NODES // NB
    row_spec = pl.BlockSpec((T, NB, HID), lambda i: (0, i, 0))
    vec_spec = pl.BlockSpec((NB, T), lambda i: (i, 0))

    def full(shape):
        return pl.BlockSpec(shape, lambda i: tuple(0 for _ in shape))

    out_shapes = (
        jax.ShapeDtypeStruct((T, N_NODES, HID), jnp.float32),
        jax.ShapeDtypeStruct((T, N_NODES, HID), jnp.float32),
        jax.ShapeDtypeStruct((T, N_NODES, HID), jnp.float32),
        jax.ShapeDtypeStruct((N_NODES, T), jnp.float32),
        jax.ShapeDtypeStruct((N_NODES, T), jnp.float32),
    )
    in_specs = [
        pl.BlockSpec((T, NB, D_IN), lambda i: (0, i, 0)),
        pl.BlockSpec((T, NB, D_EXOG), lambda i: (0, i, 0)),
        full((D_IN, HID)), full((1, HID)), full((D_EXOG, HID)), full((1, HID)),
        full((HID, HID)), full((1, HID)), full((HID, HID)),
        full((3, HID, HID)), full((1, HID)), full((3, HID, HID)), full((1, HID)),
        full((HID, HID)), full((1, HID)),
        full((HID, HID)), full((1, HID)), full((1, HID)),
    ]
    out_specs = (row_spec, row_spec, row_spec, vec_spec, vec_spec)
    return pl.pallas_call(
        _stage1_body,
        grid=(nblk,),
        in_specs=in_specs,
        out_specs=out_specs,
        out_shape=out_shapes,
        interpret=interpret,
    )(x, uw, ec_Wx, ec_bx.reshape(1, HID), ec_Wu, ec_bu.reshape(1, HID),
      ec_Wo, ec_bo.reshape(1, HID), ec_Wc,
      t1_W, t1_b.reshape(1, HID), t2_W, t2_b.reshape(1, HID),
      skip_W, skip_b.reshape(1, HID), gat_W,
      a_src.reshape(1, HID), a_dst.reshape(1, HID))


def _stage2_body(num_ref, den_ref, h_ref, z_ref, uh_ref,
                 gat_b, ro_Wx, ro_bx, ro_Wu, ro_bu, ro_Wo, ro_bo, ro_Wc,
                 out_ref):
    num = num_ref[0] + num_ref[1]
    den = den_ref[...]
    wo = ro_Wo[0, 0]
    wc = ro_Wc[0, 0]
    bx = ro_bx[0, 0]
    bu = ro_bu[0, 0]
    bo = ro_bo[0, 0]
    for t in range(T):
        s = _relu(num[t] / (den[:, t][:, None] + 1e-16) + gat_b[...])
        hf = h_ref[t] + z_ref[t] + s
        cond = jnp.sum(uh_ref[t] * ro_Wu[...], axis=-1) + bu
        pre = _relu(jnp.sum(hf * ro_Wx[...], axis=-1) + bx + cond)
        out_ref[:, t] = pre * wo + bo + cond * wc


def _stage2(num, den, h, z, uh, gat_b, ro_Wx, ro_bx, ro_Wu, ro_bu,
            ro_Wo, ro_bo, ro_Wc, interpret=False):
    nblk = N_NODES // NB

    def full(shape):
        return pl.BlockSpec(shape, lambda i: tuple(0 for _ in shape))

    in_specs = [
        pl.BlockSpec((2, T, NB, HID), lambda i: (0, 0, i, 0)),
        pl.BlockSpec((NB, T), lambda i: (i, 0)),
        pl.BlockSpec((T, NB, HID), lambda i: (0, i, 0)),
        pl.BlockSpec((T, NB, HID), lambda i: (0, i, 0)),
        pl.BlockSpec((T, NB, D_EXOG), lambda i: (0, i, 0)),
        full((1, HID)),
        full((1, HID)),
        full((1, 1)), full((1, D_EXOG)), full((1, 1)),
        full((1, 1)), full((1, 1)), full((1, 1)),
    ]
    return pl.pallas_call(
        _stage2_body,
        grid=(nblk,),
        in_specs=in_specs,
        out_specs=pl.BlockSpec((NB, T), lambda i: (i, 0)),
        out_shape=jax.ShapeDtypeStruct((N_NODES, T), jnp.float32),
        interpret=interpret,
    )(num, den, h, z, uh, gat_b.reshape(1, HID), ro_Wx.reshape(1, HID),
      ro_bx.reshape(1, 1), ro_Wu.reshape(1, D_EXOG), ro_bu.reshape(1, 1),
      ro_Wo.reshape(1, 1), ro_bo.reshape(1, 1), ro_Wc.reshape(1, 1))


NW = 32
NQ = 2
GQ = 128
G = NQ * GQ
CHUNK = 5120
NBATCH = CHUNK // G
EP = NW * CHUNK
NP = N_NODES + 8
ROWS_PER_TILE = NPAD // 16


def _sc_edge_body(src_hbm, dst_hbm, stab_hbm, dtab_hbm, gmax_hbm, hgt_hbm,
                  num_hbm, den_hbm,
                  sidx, didx, stab, dtab, gvec, wbuf, gidx, rows, den,
                  num_sh, sg00, sg01, sg10, sg11, sem2):
    sc = lax.axis_index("c")
    tid = lax.axis_index("s")
    wid = sc * 16 + tid
    zeros16 = jnp.zeros((16,), jnp.float32)
    izeros = jnp.zeros((16,), jnp.int32)
    sg = ((sg00, sg01), (sg10, sg11))

    pltpu.sync_copy(src_hbm.at[wid], sidx)
    pltpu.sync_copy(dst_hbm.at[wid], didx)

    for t in range(T):
        pltpu.sync_copy(stab_hbm.at[t], stab)
        pltpu.sync_copy(dtab_hbm.at[t], dtab)
        pltpu.sync_copy(gmax_hbm.at[t], gvec)
        g0 = gvec[...][0]

        @plsc.parallel_loop(0, NPAD // 16, unroll=8)
        def _zd(i):
            den[pl.ds(i * 16, 16)] = zeros16

        @plsc.parallel_loop(0, GQ, unroll=4)
        def _zr(r):
            for c in range(HID // 16):
                rows[0, 0, r, pl.ds(c * 16, 16)] = zeros16
        for jz in range(ROWS_PER_TILE // GQ):
            pltpu.sync_copy(
                rows.at[0, 0],
                num_sh.at[pl.ds(tid * ROWS_PER_TILE + jz * GQ, GQ)])
        plsc.subcore_barrier()

        def _pa(j, bank):
            for q in range(NQ):
                @plsc.parallel_loop(0, GQ // 16, unroll=4)
                def _paq(g):
                    off = g * 16
                    sv = sidx[j, q, pl.ds(off, 16)]
                    dv = didx[j, q, pl.ds(off, 16)]
                    a = plsc.load_gather(stab, [sv])
                    b = plsc.load_gather(dtab, [dv])
                    e = a + b
                    e = jnp.where(e >= 0.0, e, 0.2 * e)
                    ub = b + g0
                    ub = jnp.where(ub >= 0.0, ub, 0.2 * ub)
                    w = jnp.exp(e - ub)
                    wbuf[bank, q, pl.ds(off, 16)] = w
                    plsc.addupdate_scatter(den, [dv], w)
                    gidx[bank, q, pl.ds(off, 16)] = (
                        jnp.minimum(sv, N_NODES - 1) + t * N_NODES)

        def _gat(bank):
            for q in range(NQ):
                pltpu.async_copy(hgt_hbm.at[gidx.at[bank, q]],
                                 rows.at[bank, q], sg[bank][q])

        def _wait_gat(bank):
            for q in range(NQ):
                pltpu.make_async_copy(hgt_hbm.at[gidx.at[bank, q]],
                                      rows.at[bank, q], sg[bank][q]).wait()

        def _scale(bank):
            for q in range(NQ):
                bv = izeros + bank
                qv = izeros + q

                @plsc.parallel_loop(0, GQ, unroll=8)
                def _sr(r):
                    wsplat = plsc.load_gather(wbuf, [bv, qv, izeros + r])
                    for c in range(HID // 16):
                        rows[bank, q, r, pl.ds(c * 16, 16)] = (
                            wsplat * rows[bank, q, r, pl.ds(c * 16, 16)])

        def _scat(j, bank):
            for q in range(NQ):
                pltpu.async_copy(rows.at[bank, q], num_sh.at[didx.at[j, q]],
                                 sem2, add=True)

        def _drain_scat(j, bank):
            for q in range(NQ):
                pltpu.make_async_copy(rows.at[bank, q],
                                      num_sh.at[didx.at[j, q]], sem2).wait()

        _pa(0, 0)
        _gat(0)

        def _jj(jj, _):
            j0 = 2 * jj
            j1 = j0 + 1
            _pa(j1, 1)

            @pl.when(jj > 0)
            def _():
                _drain_scat(j0 - 1, 1)
            _gat(1)
            _wait_gat(0)
            _scale(0)
            _scat(j0, 0)

            @pl.when(jj < NBATCH // 2 - 1)
            def _():
                _pa(j0 + 2, 0)
            _wait_gat(1)
            _scale(1)
            _drain_scat(j0, 0)

            @pl.when(jj < NBATCH // 2 - 1)
            def _():
                _gat(0)
            _scat(j1, 1)
            return _
        lax.fori_loop(0, NBATCH // 2, _jj, None)
        _drain_scat(NBATCH - 1, 1)

        plsc.subcore_barrier()
        pltpu.sync_copy(
            num_sh.at[pl.ds(tid * ROWS_PER_TILE, ROWS_PER_TILE)],
            num_hbm.at[sc, t, pl.ds(tid * ROWS_PER_TILE, ROWS_PER_TILE)])
        pltpu.sync_copy(den, den_hbm.at[wid, t])


def _gat_edges_sc(hg, ssrc, sdst, gmax, src, dst):
    npad_e = EP - N_EDGES
    srcp = jnp.concatenate(
        [src, jnp.full((npad_e,), N_NODES, jnp.int32)]).reshape(
            NW, NBATCH, NQ, GQ)
    dstp = jnp.concatenate(
        [dst, jnp.zeros((npad_e,), jnp.int32)]).reshape(NW, NBATCH, NQ, GQ)
    neg = jnp.full((T, NP - N_NODES), -1e30, jnp.float32)
    stab = jnp.concatenate([ssrc.T, neg], axis=1)
    dtab = jnp.concatenate([sdst.T, neg], axis=1)
    gmax16 = jnp.broadcast_to(gmax.reshape(T, 1), (T, 16))
    hgt = hg.reshape(T * N_NODES, HID)

    mesh = plsc.VectorSubcoreMesh(core_axis_name="c", subcore_axis_name="s")
    num, den = pl.kernel(
        _sc_edge_body,
        out_type=(
            jax.ShapeDtypeStruct((2, T, NPAD, HID), jnp.float32),
            jax.ShapeDtypeStruct((NW, T, NPAD), jnp.float32),
        ),
        mesh=mesh,
        compiler_params=pltpu.CompilerParams(
            needs_layout_passes=False, use_tc_tiling_on_sc=False),
        scratch_types=[
            pltpu.VMEM((NBATCH, NQ, GQ), jnp.int32),
            pltpu.VMEM((NBATCH, NQ, GQ), jnp.int32),
            pltpu.VMEM((NP,), jnp.float32),
            pltpu.VMEM((NP,), jnp.float32),
            pltpu.VMEM((16,), jnp.float32),
            pltpu.VMEM((2, NQ, GQ), jnp.float32),
            pltpu.VMEM((2, NQ, GQ), jnp.int32),
            pltpu.VMEM((2, NQ, GQ, HID), jnp.float32),
            pltpu.VMEM((NPAD,), jnp.float32),
            pltpu.VMEM_SHARED((NPAD, HID), jnp.float32),
            pltpu.SemaphoreType.DMA,
            pltpu.SemaphoreType.DMA,
            pltpu.SemaphoreType.DMA,
            pltpu.SemaphoreType.DMA,
            pltpu.SemaphoreType.DMA,
        ],
    )(srcp, dstp, stab, dtab, gmax16, hgt)
    return num, den


def _gat_edges_xla(hg, ssrc, sdst, gmax, src, dst):
    num = jnp.zeros((T, NPAD, HID), jnp.float32)
    den = jnp.zeros((T, NPAD), jnp.float32)
    for t in range(T):
        e = ssrc[src, t] + sdst[dst, t]
        e = jnp.where(e >= 0, e, 0.2 * e)
        ubv = sdst[dst, t] + gmax[t]
        ubv = jnp.where(ubv >= 0, ubv, 0.2 * ubv)
        w = jnp.exp(e - ubv)
        den = den.at[t].set(
            jax.ops.segment_sum(w, dst, num_segments=NPAD))
        msg = hg[t, src] * w[:, None]
        num = num.at[t].set(
            jax.ops.segment_sum(msg, dst, num_segments=NPAD))
    num2 = jnp.stack([num, jnp.zeros_like(num)])
    return num2, den[None]


def kernel(x, u_w, u_h, edge_index, ec_Wx, ec_bx, ec_Wu, ec_bu, ec_Wo,
           ec_bo, ec_Wc, t1_W, t1_b, t2_W, t2_b, skip_W, skip_b, gat_W,
           gat_b, a_src, a_dst, ro_Wx, ro_bx, ro_Wu, ro_bu, ro_Wo, ro_bo,
           ro_Wc, interpret=False):
    h, z, hg, ssrc, sdst = _stage1(
        x[0], u_w[0], ec_Wx, ec_bx, ec_Wu, ec_bu, ec_Wo, ec_bo, ec_Wc,
        t1_W, t1_b, t2_W, t2_b, skip_W, skip_b, gat_W, a_src, a_dst,
        interpret=interpret)

    gmax = jnp.max(ssrc, axis=0)

    src = edge_index[0]
    dst = edge_index[1]
    if interpret:
        num2, den = _gat_edges_xla(hg, ssrc, sdst, gmax, src, dst)
    else:
        num2, den = _gat_edges_sc(hg, ssrc, sdst, gmax, src, dst)

    denT = jnp.sum(den, axis=0)[:, :N_NODES].T

    out = _stage2(num2, denT, h, z, u_h[0], gat_b, ro_Wx, ro_bx, ro_Wu,
                  ro_bu, ro_Wo, ro_bo, ro_Wc, interpret=interpret)
    return out.T.reshape(1, T, N_NODES, 1)


# speedup vs baseline: 1.5869x; 1.3524x over previous
import functools

import jax
import jax.numpy as jnp
from jax import lax
from jax.experimental import pallas as pl
from jax.experimental.pallas import tpu as pltpu
from jax.experimental.pallas import tpu_sc as plsc

N_NODES = 10000
N_EDGES = 160000
T = 4
D_IN = 8
D_EXOG = 16
HID = 64

NB = 2000
NPAD = 10240


def _relu(v):
    return jnp.maximum(v, 0.0)


def _dot(a, b):
    return jnp.dot(a, b, preferred_element_type=jnp.float32)


def _stage1_body(x_ref, uw_ref,
                 ec_Wx, ec_bx, ec_Wu, ec_bu, ec_Wo, ec_bo, ec_Wc,
                 t1_W, t1_b, t2_W, t2_b, skip_W, skip_b,
                 gat_W, a_src, a_dst,
                 h_out, z_out, hg_out, ssrc_out, sdst_out):
    x2 = x_ref[...].reshape(T * NB, D_IN)
    u2 = uw_ref[...].reshape(T * NB, D_EXOG)
    cond = _dot(u2, ec_Wu[...]) + ec_bu[...]
    pre = _relu(_dot(x2, ec_Wx[...]) + ec_bx[...] + cond)
    h2 = _dot(pre, ec_Wo[...]) + ec_bo[...] + _dot(cond, ec_Wc[...])
    h4 = h2.reshape(T, NB, HID)
    h_out[...] = h4

    z1 = []
    for t in range(T):
        acc = t1_b[...]
        for k in range(3):
            j = t - 2 + k
            if 0 <= j < T:
                acc = acc + _dot(h4[j], t1_W[k])
        z1.append(_relu(acc))
    z2 = []
    for t in range(T):
        acc = t2_b[...]
        for k in range(3):
            j = t - 2 + k
            if 0 <= j < T:
                acc = acc + _dot(z1[j], t2_W[k])
        z2.append(_relu(acc))
    for t in range(T):
        z_t = _dot(h4[t], skip_W[...]) + skip_b[...] + z2[t]
        z_out[t] = z_t
        hg_t = _dot(z_t, gat_W[...])
        hg_out[t] = hg_t
        ssrc_out[:, t] = jnp.sum(hg_t * a_src[...], axis=-1)
        sdst_out[:, t] = jnp.sum(hg_t * a_dst[...], axis=-1)


def _stage1(x, uw, ec_Wx, ec_bx, ec_Wu, ec_bu, ec_Wo, ec_bo, ec_Wc,
            t1_W, t1_b, t2_W, t2_b, skip_W, skip_b, gat_W, a_src, a_dst,
            interpret=False):
    nblk = N_NODES // NB
    row_spec = pl.BlockSpec((T, NB, HID), lambda i: (0, i, 0))
    vec_spec = pl.BlockSpec((NB, T), lambda i: (i, 0))

    def full(shape):
        return pl.BlockSpec(shape, lambda i: tuple(0 for _ in shape))

    out_shapes = (
        jax.ShapeDtypeStruct((T, N_NODES, HID), jnp.float32),
        jax.ShapeDtypeStruct((T, N_NODES, HID), jnp.float32),
        jax.ShapeDtypeStruct((T, N_NODES, HID), jnp.float32),
        jax.ShapeDtypeStruct((N_NODES, T), jnp.float32),
        jax.ShapeDtypeStruct((N_NODES, T), jnp.float32),
    )
    in_specs = [
        pl.BlockSpec((T, NB, D_IN), lambda i: (0, i, 0)),
        pl.BlockSpec((T, NB, D_EXOG), lambda i: (0, i, 0)),
        full((D_IN, HID)), full((1, HID)), full((D_EXOG, HID)), full((1, HID)),
        full((HID, HID)), full((1, HID)), full((HID, HID)),
        full((3, HID, HID)), full((1, HID)), full((3, HID, HID)), full((1, HID)),
        full((HID, HID)), full((1, HID)),
        full((HID, HID)), full((1, HID)), full((1, HID)),
    ]
    out_specs = (row_spec, row_spec, row_spec, vec_spec, vec_spec)
    return pl.pallas_call(
        _stage1_body,
        grid=(nblk,),
        in_specs=in_specs,
        out_specs=out_specs,
        out_shape=out_shapes,
        interpret=interpret,
    )(x, uw, ec_Wx, ec_bx.reshape(1, HID), ec_Wu, ec_bu.reshape(1, HID),
      ec_Wo, ec_bo.reshape(1, HID), ec_Wc,
      t1_W, t1_b.reshape(1, HID), t2_W, t2_b.reshape(1, HID),
      skip_W, skip_b.reshape(1, HID), gat_W,
      a_src.reshape(1, HID), a_dst.reshape(1, HID))


def _stage2_body(num_ref, den_ref, h_ref, z_ref, uh_ref,
                 gat_b, ro_Wx, ro_bx, ro_Wu, ro_bu, ro_Wo, ro_bo, ro_Wc,
                 out_ref):
    num = num_ref[...]
    den = den_ref[...]
    wo = ro_Wo[0, 0]
    wc = ro_Wc[0, 0]
    bx = ro_bx[0, 0]
    bu = ro_bu[0, 0]
    bo = ro_bo[0, 0]
    for t in range(T):
        s = _relu(num[t] / (den[:, t][:, None] + 1e-16) + gat_b[...])
        hf = h_ref[t] + z_ref[t] + s
        cond = jnp.sum(uh_ref[t] * ro_Wu[...], axis=-1) + bu
        pre = _relu(jnp.sum(hf * ro_Wx[...], axis=-1) + bx + cond)
        out_ref[:, t] = pre * wo + bo + cond * wc


def _stage2(num, den, h, z, uh, gat_b, ro_Wx, ro_bx, ro_Wu, ro_bu,
            ro_Wo, ro_bo, ro_Wc, interpret=False):
    nblk = N_NODES // NB

    def full(shape):
        return pl.BlockSpec(shape, lambda i: tuple(0 for _ in shape))

    in_specs = [
        pl.BlockSpec((T, NB, HID), lambda i: (0, i, 0)),
        pl.BlockSpec((NB, T), lambda i: (i, 0)),
        pl.BlockSpec((T, NB, HID), lambda i: (0, i, 0)),
        pl.BlockSpec((T, NB, HID), lambda i: (0, i, 0)),
        pl.BlockSpec((T, NB, D_EXOG), lambda i: (0, i, 0)),
        full((1, HID)),
        full((1, HID)),
        full((1, 1)), full((1, D_EXOG)), full((1, 1)),
        full((1, 1)), full((1, 1)), full((1, 1)),
    ]
    return pl.pallas_call(
        _stage2_body,
        grid=(nblk,),
        in_specs=in_specs,
        out_specs=pl.BlockSpec((NB, T), lambda i: (i, 0)),
        out_shape=jax.ShapeDtypeStruct((N_NODES, T), jnp.float32),
        interpret=interpret,
    )(num, den, h, z, uh, gat_b.reshape(1, HID), ro_Wx.reshape(1, HID),
      ro_bx.reshape(1, 1), ro_Wu.reshape(1, D_EXOG), ro_bu.reshape(1, 1),
      ro_Wo.reshape(1, 1), ro_bo.reshape(1, 1), ro_Wc.reshape(1, 1))


NW = 32
NQ = 2
GQ = 128
G = NQ * GQ
CHUNK = 10240
NBATCH = CHUNK // G
EP = 16 * CHUNK
NP = N_NODES + 8
ROWS_PER_TILE = NPAD // 16


def _sc_edge_body(src_hbm, dst_hbm, stab_hbm, dtab_hbm, gmax_hbm, hgt_hbm,
                  num_hbm, den_hbm,
                  sidx, didx, stab, dtab, gvec, wbuf, gidx, rows, den,
                  num_sh, sg00, sg01, sg10, sg11, sem2):
    sc = lax.axis_index("c")
    tid = lax.axis_index("s")
    wid = sc * 16 + tid
    zeros16 = jnp.zeros((16,), jnp.float32)
    izeros = jnp.zeros((16,), jnp.int32)
    sg = ((sg00, sg01), (sg10, sg11))

    pltpu.sync_copy(src_hbm.at[tid], sidx)
    pltpu.sync_copy(dst_hbm.at[tid], didx)

    for tt in range(T // 2):
        t = sc * (T // 2) + tt
        pltpu.sync_copy(stab_hbm.at[t], stab)
        pltpu.sync_copy(dtab_hbm.at[t], dtab)
        pltpu.sync_copy(gmax_hbm.at[t], gvec)
        g0 = gvec[...][0]

        @plsc.parallel_loop(0, NPAD // 16, unroll=8)
        def _zd(i):
            den[pl.ds(i * 16, 16)] = zeros16

        @plsc.parallel_loop(0, GQ, unroll=4)
        def _zr(r):
            for c in range(HID // 16):
                rows[0, 0, r, pl.ds(c * 16, 16)] = zeros16
        for jz in range(ROWS_PER_TILE // GQ):
            pltpu.sync_copy(
                rows.at[0, 0],
                num_sh.at[pl.ds(tid * ROWS_PER_TILE + jz * GQ, GQ)])
        plsc.subcore_barrier()

        def _pa(j, bank):
            for q in range(NQ):
                @plsc.parallel_loop(0, GQ // 16, unroll=4)
                def _paq(g):
                    off = g * 16
                    sv = sidx[j, q, pl.ds(off, 16)]
                    dv = didx[j, q, pl.ds(off, 16)]
                    a = plsc.load_gather(stab, [sv])
                    b = plsc.load_gather(dtab, [dv])
                    e = a + b
                    e = jnp.where(e >= 0.0, e, 0.2 * e)
                    ub = b + g0
                    ub = jnp.where(ub >= 0.0, ub, 0.2 * ub)
                    w = jnp.exp(e - ub)
                    wbuf[bank, q, pl.ds(off, 16)] = w
                    plsc.addupdate_scatter(den, [dv], w)
                    gidx[bank, q, pl.ds(off, 16)] = (
                        jnp.minimum(sv, N_NODES - 1) + t * N_NODES)

        def _gat(bank):
            for q in range(NQ):
                pltpu.async_copy(hgt_hbm.at[gidx.at[bank, q]],
                                 rows.at[bank, q], sg[bank][q])

        def _wait_gat(bank):
            for q in range(NQ):
                pltpu.make_async_copy(hgt_hbm.at[gidx.at[bank, q]],
                                      rows.at[bank, q], sg[bank][q]).wait()

        def _scale(bank):
            for q in range(NQ):
                bv = izeros + bank
                qv = izeros + q

                @plsc.parallel_loop(0, GQ, unroll=8)
                def _sr(r):
                    wsplat = plsc.load_gather(wbuf, [bv, qv, izeros + r])
                    for c in range(HID // 16):
                        rows[bank, q, r, pl.ds(c * 16, 16)] = (
                            wsplat * rows[bank, q, r, pl.ds(c * 16, 16)])

        def _scat(j, bank):
            for q in range(NQ):
                pltpu.async_copy(rows.at[bank, q], num_sh.at[didx.at[j, q]],
                                 sem2, add=True)

        def _drain_scat(j, bank):
            for q in range(NQ):
                pltpu.make_async_copy(rows.at[bank, q],
                                      num_sh.at[didx.at[j, q]], sem2).wait()

        _pa(0, 0)
        _gat(0)

        def _jj(jj, _):
            j0 = 2 * jj
            j1 = j0 + 1
            _pa(j1, 1)

            @pl.when(jj > 0)
            def _():
                _drain_scat(j0 - 1, 1)
            _gat(1)
            _wait_gat(0)
            _scale(0)
            _scat(j0, 0)

            @pl.when(jj < NBATCH // 2 - 1)
            def _():
                _pa(j0 + 2, 0)
            _wait_gat(1)
            _scale(1)
            _drain_scat(j0, 0)

            @pl.when(jj < NBATCH // 2 - 1)
            def _():
                _gat(0)
            _scat(j1, 1)
            return _
        lax.fori_loop(0, NBATCH // 2, _jj, None)
        _drain_scat(NBATCH - 1, 1)

        plsc.subcore_barrier()
        pltpu.sync_copy(
            num_sh.at[pl.ds(tid * ROWS_PER_TILE, ROWS_PER_TILE)],
            num_hbm.at[t, pl.ds(tid * ROWS_PER_TILE, ROWS_PER_TILE)])
        pltpu.sync_copy(den, den_hbm.at[tid, t])


def _gat_edges_sc(hg, ssrc, sdst, gmax, src, dst):
    npad_e = EP - N_EDGES
    srcp = jnp.concatenate(
        [src, jnp.full((npad_e,), N_NODES, jnp.int32)]).reshape(
            16, NBATCH, NQ, GQ)
    dstp = jnp.concatenate(
        [dst, jnp.zeros((npad_e,), jnp.int32)]).reshape(16, NBATCH, NQ, GQ)
    neg = jnp.full((T, NP - N_NODES), -1e30, jnp.float32)
    stab = jnp.concatenate([ssrc.T, neg], axis=1)
    dtab = jnp.concatenate([sdst.T, neg], axis=1)
    gmax16 = jnp.broadcast_to(gmax.reshape(T, 1), (T, 16))
    hgt = hg.reshape(T * N_NODES, HID)

    mesh = plsc.VectorSubcoreMesh(core_axis_name="c", subcore_axis_name="s")
    num, den = pl.kernel(
        _sc_edge_body,
        out_type=(
            jax.ShapeDtypeStruct((T, NPAD, HID), jnp.float32),
            jax.ShapeDtypeStruct((16, T, NPAD), jnp.float32),
        ),
        mesh=mesh,
        compiler_params=pltpu.CompilerParams(
            needs_layout_passes=False, use_tc_tiling_on_sc=False),
        scratch_types=[
            pltpu.VMEM((NBATCH, NQ, GQ), jnp.int32),
            pltpu.VMEM((NBATCH, NQ, GQ), jnp.int32),
            pltpu.VMEM((NP,), jnp.float32),
            pltpu.VMEM((NP,), jnp.float32),
            pltpu.VMEM((16,), jnp.float32),
            pltpu.VMEM((2, NQ, GQ), jnp.float32),
            pltpu.VMEM((2, NQ, GQ), jnp.int32),
            pltpu.VMEM((2, NQ, GQ, HID), jnp.float32),
            pltpu.VMEM((NPAD,), jnp.float32),
            pltpu.VMEM_SHARED((NPAD, HID), jnp.float32),
            pltpu.SemaphoreType.DMA,
            pltpu.SemaphoreType.DMA,
            pltpu.SemaphoreType.DMA,
            pltpu.SemaphoreType.DMA,
            pltpu.SemaphoreType.DMA,
        ],
    )(srcp, dstp, stab, dtab, gmax16, hgt)
    return num, den


def _gat_edges_xla(hg, ssrc, sdst, gmax, src, dst):
    num = jnp.zeros((T, NPAD, HID), jnp.float32)
    den = jnp.zeros((T, NPAD), jnp.float32)
    for t in range(T):
        e = ssrc[src, t] + sdst[dst, t]
        e = jnp.where(e >= 0, e, 0.2 * e)
        ubv = sdst[dst, t] + gmax[t]
        ubv = jnp.where(ubv >= 0, ubv, 0.2 * ubv)
        w = jnp.exp(e - ubv)
        den = den.at[t].set(
            jax.ops.segment_sum(w, dst, num_segments=NPAD))
        msg = hg[t, src] * w[:, None]
        num = num.at[t].set(
            jax.ops.segment_sum(msg, dst, num_segments=NPAD))
    return num, den[None]


def kernel(x, u_w, u_h, edge_index, ec_Wx, ec_bx, ec_Wu, ec_bu, ec_Wo,
           ec_bo, ec_Wc, t1_W, t1_b, t2_W, t2_b, skip_W, skip_b, gat_W,
           gat_b, a_src, a_dst, ro_Wx, ro_bx, ro_Wu, ro_bu, ro_Wo, ro_bo,
           ro_Wc, interpret=False):
    h, z, hg, ssrc, sdst = _stage1(
        x[0], u_w[0], ec_Wx, ec_bx, ec_Wu, ec_bu, ec_Wo, ec_bo, ec_Wc,
        t1_W, t1_b, t2_W, t2_b, skip_W, skip_b, gat_W, a_src, a_dst,
        interpret=interpret)

    gmax = jnp.max(ssrc, axis=0)

    src = edge_index[0]
    dst = edge_index[1]
    if interpret:
        num2, den = _gat_edges_xla(hg, ssrc, sdst, gmax, src, dst)
    else:
        num2, den = _gat_edges_sc(hg, ssrc, sdst, gmax, src, dst)

    denT = jnp.sum(den, axis=0)[:, :N_NODES].T

    out = _stage2(num2, denT, h, z, u_h[0], gat_b, ro_Wx, ro_bx, ro_Wu,
                  ro_bu, ro_Wo, ro_bo, ro_Wc, interpret=interpret)
    return out.T.reshape(1, T, N_NODES, 1)
